# Initial kernel scaffold; baseline (speedup 1.0000x reference)
import jax
import jax.numpy as jnp
from jax.experimental import pallas as pl


def kernel(z, mu, log_var, log_alpha):
    raise NotImplementedError("write your pallas kernel here")


# speedup vs baseline: 8.8300x; 8.8300x over previous
import jax
import jax.numpy as jnp
from jax.experimental import pallas as pl


def _gmm_logprob_kernel(z_ref, mu_ref, log_var_ref, la_ref, out_ref):
    z = z_ref[...]
    mu = mu_ref[...]
    log_var = log_var_ref[...]
    la = la_ref[...]

    var = jax.nn.softplus(log_var)
    inv_var = 1.0 / var
    log_det = jnp.sum(jnp.log(var), axis=1, keepdims=True)
    d = jnp.sum(mu * mu * inv_var, axis=1, keepdims=True)

    la_max = jnp.max(la)
    log_norm = la_max + jnp.log(jnp.sum(jnp.exp(la - la_max)))

    c = (la - log_norm) - 0.5 * (log_det + d)
    w2 = jnp.concatenate([-0.5 * inv_var, c], axis=1)
    one = jnp.ones((z.shape[0], 1), jnp.float32)
    z2e = jnp.concatenate([z * z, one], axis=1)

    g1 = jax.lax.dot_general(z, mu * inv_var, (((1,), (1,)), ((), ())),
                             preferred_element_type=jnp.float32)
    g2 = jax.lax.dot_general(z2e, w2, (((1,), (1,)), ((), ())),
                             preferred_element_type=jnp.float32)

    t = g1 + g2
    m = jnp.max(t, axis=1, keepdims=True)
    out_ref[...] = m + jnp.log(jnp.sum(jnp.exp(t - m), axis=1, keepdims=True))


def kernel(z, mu, log_var, log_alpha):
    B, F = z.shape
    K = mu.shape[0]
    BB = 256
    la = jnp.concatenate([log_alpha, jnp.zeros((1,), log_alpha.dtype)])
    out = pl.pallas_call(
        _gmm_logprob_kernel,
        grid=(B // BB,),
        in_specs=[
            pl.BlockSpec((BB, F), lambda i: (i, 0)),
            pl.BlockSpec((K, F), lambda i: (0, 0)),
            pl.BlockSpec((K, F), lambda i: (0, 0)),
            pl.BlockSpec((K, 1), lambda i: (0, 0)),
        ],
        out_specs=pl.BlockSpec((BB, 1), lambda i: (i, 0)),
        out_shape=jax.ShapeDtypeStruct((B, 1), jnp.float32),
    )(z, mu, log_var, la.reshape(K, 1))
    return out.reshape(B)


# speedup vs baseline: 11.4713x; 1.2991x over previous
import jax
import jax.numpy as jnp
from jax.experimental import pallas as pl


def _gmm_logprob_kernel(z_ref, mu_ref, log_var_ref, la_ref, out_ref):
    z = z_ref[...]
    mu = mu_ref[...]
    log_var = log_var_ref[...]
    la = la_ref[...]

    var = jax.nn.softplus(log_var)
    inv_var = 1.0 / var
    log_det = jnp.sum(jnp.log(var), axis=1, keepdims=True)
    d = jnp.sum(mu * mu * inv_var, axis=1, keepdims=True)

    la_max = jnp.max(la)
    log_norm = la_max + jnp.log(jnp.sum(jnp.exp(la - la_max)))

    c = (la - log_norm) - 0.5 * (log_det + d)
    w2 = jnp.concatenate([-0.5 * inv_var, c], axis=1)
    one = jnp.ones((z.shape[0], 1), jnp.float32)
    z2e = jnp.concatenate([z * z, one], axis=1)

    g1 = jax.lax.dot_general(z, mu * inv_var, (((1,), (1,)), ((), ())),
                             preferred_element_type=jnp.float32)
    g2 = jax.lax.dot_general(z2e, w2, (((1,), (1,)), ((), ())),
                             preferred_element_type=jnp.float32)

    t = g1 + g2
    m = jnp.max(t, axis=1, keepdims=True)
    out_ref[...] = m + jnp.log(jnp.sum(jnp.exp(t - m), axis=1, keepdims=True))


def kernel(z, mu, log_var, log_alpha):
    B, F = z.shape
    K = mu.shape[0]
    BB = 1024
    la = jnp.concatenate([log_alpha, jnp.zeros((1,), log_alpha.dtype)])
    out = pl.pallas_call(
        _gmm_logprob_kernel,
        grid=(B // BB,),
        in_specs=[
            pl.BlockSpec((BB, F), lambda i: (i, 0)),
            pl.BlockSpec((K, F), lambda i: (0, 0)),
            pl.BlockSpec((K, F), lambda i: (0, 0)),
            pl.BlockSpec((K, 1), lambda i: (0, 0)),
        ],
        out_specs=pl.BlockSpec((BB, 1), lambda i: (i, 0)),
        out_shape=jax.ShapeDtypeStruct((B, 1), jnp.float32),
    )(z, mu, log_var, la.reshape(K, 1))
    return out.reshape(B)


# speedup vs baseline: 11.5590x; 1.0076x over previous
import jax
import jax.numpy as jnp
from jax.experimental import pallas as pl


def _gmm_logprob_kernel(z_ref, mu_ref, log_var_ref, la_ref, out_ref):
    z = z_ref[...]
    mu = mu_ref[...]
    log_var = log_var_ref[...]
    la = jnp.concatenate(
        [la_ref[...], jnp.zeros((1, 1), jnp.float32)], axis=0)

    var = jax.nn.softplus(log_var)
    inv_var = 1.0 / var
    log_det = jnp.sum(jnp.log(var), axis=1, keepdims=True)
    d = jnp.sum(mu * mu * inv_var, axis=1, keepdims=True)

    la_max = jnp.max(la)
    log_norm = la_max + jnp.log(jnp.sum(jnp.exp(la - la_max)))

    c = (la - log_norm) - 0.5 * (log_det + d)
    w2 = jnp.concatenate([-0.5 * inv_var, c], axis=1)
    one = jnp.ones((z.shape[0], 1), jnp.float32)
    z2e = jnp.concatenate([z * z, one], axis=1)

    g1 = jax.lax.dot_general(z, mu * inv_var, (((1,), (1,)), ((), ())),
                             preferred_element_type=jnp.float32)
    g2 = jax.lax.dot_general(z2e, w2, (((1,), (1,)), ((), ())),
                             preferred_element_type=jnp.float32)

    t = g1 + g2
    m = jnp.max(t, axis=1, keepdims=True)
    out_ref[...] = m + jnp.log(jnp.sum(jnp.exp(t - m), axis=1, keepdims=True))


def kernel(z, mu, log_var, log_alpha):
    B, F = z.shape
    K = mu.shape[0]
    BB = 1024
    out = pl.pallas_call(
        _gmm_logprob_kernel,
        grid=(B // BB,),
        in_specs=[
            pl.BlockSpec((BB, F), lambda i: (i, 0)),
            pl.BlockSpec((K, F), lambda i: (0, 0)),
            pl.BlockSpec((K, F), lambda i: (0, 0)),
            pl.BlockSpec((K - 1, 1), lambda i: (0, 0)),
        ],
        out_specs=pl.BlockSpec((BB, 1), lambda i: (i, 0)),
        out_shape=jax.ShapeDtypeStruct((B, 1), jnp.float32),
    )(z, mu, log_var, log_alpha.reshape(K - 1, 1))
    return out.reshape(B)


# speedup vs baseline: 11.7206x; 1.0140x over previous
import jax
import jax.numpy as jnp
from jax.experimental import pallas as pl


def _gmm_logprob_kernel(z_ref, mu_ref, log_var_ref, la_ref, out_ref):
    z = z_ref[...]
    mu = mu_ref[...]
    log_var = log_var_ref[...]
    la = jnp.concatenate(
        [la_ref[...], jnp.zeros((1, 1), jnp.float32)], axis=0)

    var = jax.nn.softplus(log_var)
    inv_var = 1.0 / var
    log_det = jnp.sum(jnp.log(var), axis=1, keepdims=True)
    d = jnp.sum(mu * mu * inv_var, axis=1, keepdims=True)

    la_max = jnp.max(la)
    log_norm = la_max + jnp.log(jnp.sum(jnp.exp(la - la_max)))

    c = (la - log_norm) - 0.5 * (log_det + d)
    w = jnp.concatenate([mu * inv_var, -0.5 * inv_var], axis=1)
    ze = jnp.concatenate([z, z * z], axis=1)

    g = jax.lax.dot_general(ze, w, (((1,), (1,)), ((), ())),
                            preferred_element_type=jnp.float32)
    t = g + jnp.transpose(c)
    m = jnp.max(t, axis=1, keepdims=True)
    out_ref[...] = m + jnp.log(jnp.sum(jnp.exp(t - m), axis=1, keepdims=True))


def kernel(z, mu, log_var, log_alpha):
    B, F = z.shape
    K = mu.shape[0]
    BB = 1024
    out = pl.pallas_call(
        _gmm_logprob_kernel,
        grid=(B // BB,),
        in_specs=[
            pl.BlockSpec((BB, F), lambda i: (i, 0)),
            pl.BlockSpec((K, F), lambda i: (0, 0)),
            pl.BlockSpec((K, F), lambda i: (0, 0)),
            pl.BlockSpec((K - 1, 1), lambda i: (0, 0)),
        ],
        out_specs=pl.BlockSpec((BB, 1), lambda i: (i, 0)),
        out_shape=jax.ShapeDtypeStruct((B, 1), jnp.float32),
    )(z, mu, log_var, log_alpha.reshape(K - 1, 1))
    return out.reshape(B)


# speedup vs baseline: 27.1509x; 2.3165x over previous
import jax
import jax.numpy as jnp
from jax.experimental import pallas as pl


def _gmm_logprob_kernel(z_ref, mu_ref, log_var_ref, la_ref, out_ref):
    z = z_ref[...]
    mu = mu_ref[...]
    log_var = log_var_ref[...]
    la_row = jnp.concatenate(
        [la_ref[...], jnp.zeros((1, 1), jnp.float32)], axis=1)

    var = jax.nn.softplus(log_var)
    inv_var = 1.0 / var
    log_det = jnp.sum(jnp.log(var), axis=1, keepdims=True)
    d = jnp.sum(mu * mu * inv_var, axis=1, keepdims=True)

    la_max = jnp.max(la_row)
    log_norm = la_max + jnp.log(jnp.sum(jnp.exp(la_row - la_max)))

    c = (jnp.transpose(la_row) - log_norm) - 0.5 * (log_det + d)
    w = jnp.concatenate([mu * inv_var, -0.5 * inv_var], axis=1)
    ze = jnp.concatenate([z, z * z], axis=1)

    tt = jax.lax.dot_general(w, ze, (((1,), (1,)), ((), ())),
                             preferred_element_type=jnp.float32)
    tt = tt + c
    m = jnp.max(tt, axis=0, keepdims=True)
    out_ref[...] = m + jnp.log(
        jnp.sum(jnp.exp(tt - m), axis=0, keepdims=True))


def kernel(z, mu, log_var, log_alpha):
    B, F = z.shape
    K = mu.shape[0]
    out = pl.pallas_call(
        _gmm_logprob_kernel,
        grid=(1,),
        in_specs=[
            pl.BlockSpec((B, F), lambda i: (0, 0)),
            pl.BlockSpec((K, F), lambda i: (0, 0)),
            pl.BlockSpec((K, F), lambda i: (0, 0)),
            pl.BlockSpec((1, K - 1), lambda i: (0, 0)),
        ],
        out_specs=pl.BlockSpec((1, B), lambda i: (0, 0)),
        out_shape=jax.ShapeDtypeStruct((1, B), jnp.float32),
    )(z, mu, log_var, log_alpha.reshape(1, K - 1))
    return out.reshape(B)
